# Initial kernel scaffold; baseline (speedup 1.0000x reference)
import jax
import jax.numpy as jnp
from jax.experimental import pallas as pl


def kernel(x, edge_index, Wl1, Wr1, b1, g1, be1, Wl2, Wr2, b2, g2, be2, Wl3, Wr3, b3, eps):
    raise NotImplementedError("write your pallas kernel here")


# speedup vs baseline: 9.5247x; 9.5247x over previous
import functools

import jax
import jax.numpy as jnp
from jax import lax
from jax.experimental import pallas as pl
from jax.experimental.pallas import tpu as pltpu
from jax.experimental.pallas import tpu_sc as plsc

_N = 10000
_E = 320000
_D = 128
_NC = 2
_NS = 16
_NW = _NC * _NS
_EPT = _E // _NW
_CH = 125
_NCHUNK = _EPT // _CH
_NPAD = 10240
_RPT = _NPAD // _NS


_G = 16
_NGRP = _NCHUNK // _G


def _seg_body(y_hbm, src_hbm, dst_hbm, agg_out, srcg, dstg, rows0, rows1,
              acc, sem0, sem1):
    c = lax.axis_index("c")
    s = lax.axis_index("s")
    blk = c * _NS + s

    def _zrow(i, _):
        for k in range(8):
            rows0[i, pl.ds(k * 16, 16)] = jnp.zeros((16,), jnp.float32)
        return 0
    lax.fori_loop(0, 128, _zrow, 0)
    for t in range(_RPT // 128):
        pltpu.sync_copy(rows0, acc.at[pl.ds(s * _RPT + t * 128, 128)])
    plsc.subcore_barrier()

    r0 = rows0.at[pl.ds(0, _CH)]
    for g in range(_NGRP):
        pltpu.sync_copy(src_hbm.at[blk, pl.ds(g * _G, _G)], srcg)
        pltpu.sync_copy(dst_hbm.at[blk, pl.ds(g * _G, _G)], dstg)

        def _step(i, _):
            j0 = i * 2
            j1 = i * 2 + 1
            cp0 = pltpu.async_copy(y_hbm.at[srcg.at[j0]], r0, sem0)
            cp1 = pltpu.async_copy(y_hbm.at[srcg.at[j1]], rows1, sem1)
            cp0.wait()
            pltpu.sync_copy(r0, acc.at[dstg.at[j0]], add=True)
            cp1.wait()
            pltpu.sync_copy(rows1, acc.at[dstg.at[j1]], add=True)
            return 0
        lax.fori_loop(0, _G // 2, _step, 0)

    plsc.subcore_barrier()

    for t in range(_RPT // 128):
        r = s * _RPT + t * 128
        pltpu.sync_copy(acc.at[pl.ds(r, 128)], rows0)
        pltpu.sync_copy(rows0, agg_out.at[c, pl.ds(r, 128)])


def _deg_body(dst_hbm, deg_out, dstv, ones, dacc):
    c = lax.axis_index("c")
    s = lax.axis_index("s")
    blk = c * _NS + s

    pltpu.sync_copy(dst_hbm.at[blk], dstv)

    def _fill(val):
        def _row(i, _):
            ones[i, :] = jnp.full((16,), val, jnp.float32)
            return 0
        lax.fori_loop(0, 128, _row, 0)
    _fill(0.0)
    for t in range(_RPT // 128):
        pltpu.sync_copy(ones, dacc.at[pl.ds(s * _RPT + t * 128, 128)])
    _fill(1.0)
    plsc.subcore_barrier()

    o = ones.at[pl.ds(0, _CH)]
    def _step(j, _):
        pltpu.sync_copy(o, dacc.at[dstv.at[j]], add=True)
        return 0
    lax.fori_loop(0, _NCHUNK, _step, 0)

    plsc.subcore_barrier()

    for t in range(_RPT // 128):
        r = s * _RPT + t * 128
        pltpu.sync_copy(dacc.at[pl.ds(r, 128)], ones)
        pltpu.sync_copy(ones, deg_out.at[c, pl.ds(r, 128)])


def _make_seg():
    mesh = plsc.VectorSubcoreMesh(core_axis_name="c", subcore_axis_name="s")
    return pl.kernel(
        _seg_body,
        out_type=[jax.ShapeDtypeStruct((_NC, _NPAD, _D), jnp.float32)],
        mesh=mesh,
        scratch_types=[
            pltpu.VMEM((_G, _CH), jnp.int32),
            pltpu.VMEM((_G, _CH), jnp.int32),
            pltpu.VMEM((128, _D), jnp.float32),
            pltpu.VMEM((_CH, _D), jnp.float32),
            pltpu.VMEM_SHARED((_NPAD, _D), jnp.float32),
            pltpu.SemaphoreType.DMA,
            pltpu.SemaphoreType.DMA,
        ],
        compiler_params=pltpu.CompilerParams(use_tc_tiling_on_sc=False),
        name="seg_sum",
    )


def _make_deg():
    mesh = plsc.VectorSubcoreMesh(core_axis_name="c", subcore_axis_name="s")
    return pl.kernel(
        _deg_body,
        out_type=[jax.ShapeDtypeStruct((_NC, _NPAD, 16), jnp.float32)],
        mesh=mesh,
        scratch_types=[
            pltpu.VMEM((_NCHUNK, _CH), jnp.int32),
            pltpu.VMEM((128, 16), jnp.float32),
            pltpu.VMEM_SHARED((_NPAD, 16), jnp.float32),
        ],
        compiler_params=pltpu.CompilerParams(use_tc_tiling_on_sc=False),
        name="deg_sum",
    )


_seg = _make_seg()
_deg = _make_deg()


def _tc0_body(x_ref, wl1t_ref, y_ref):
    y_ref[...] = jnp.dot(x_ref[...], wl1t_ref[...],
                         preferred_element_type=jnp.float32)


def _bn(h, g, be):
    m = jnp.mean(h, axis=0, keepdims=True)
    v = jnp.mean((h - m) * (h - m), axis=0, keepdims=True)
    return (h - m) / jnp.sqrt(v + 1e-5) * g + be


def _deg_from(dp_ref):
    deg = dp_ref[0, : _N, 0:1] + dp_ref[1, : _N, 0:1]
    return jnp.maximum(deg, 1.0)


def _tc1_body(sp_ref, dp_ref, x_ref, wr1t_ref, b1_ref, g1_ref, be1_ref,
              h1_ref):
    s = sp_ref[0, : _N, :] + sp_ref[1, : _N, :]
    pre = (s / _deg_from(dp_ref) + b1_ref[...]
           + jnp.dot(x_ref[...], wr1t_ref[...],
                     preferred_element_type=jnp.float32))
    h = jnp.maximum(pre, 0.0)
    h1_ref[...] = _bn(h, g1_ref[...], be1_ref[...])


def _tc2_body(sp_ref, dp_ref, h1_ref, wl2t_ref, wr2t_ref, b2_ref, g2_ref,
              be2_ref, wl3t_ref, h2_ref, y3_ref):
    s = sp_ref[0, : _N, :] + sp_ref[1, : _N, :]
    agg = s / _deg_from(dp_ref)
    pre = (jnp.dot(agg, wl2t_ref[...], preferred_element_type=jnp.float32)
           + b2_ref[...]
           + jnp.dot(h1_ref[...], wr2t_ref[...],
                     preferred_element_type=jnp.float32))
    h = jnp.maximum(pre, 0.0)
    h2 = _bn(h, g2_ref[...], be2_ref[...])
    h2_ref[...] = h2
    y3_ref[...] = jnp.dot(h2, wl3t_ref[...],
                          preferred_element_type=jnp.float32)


def _tc3_body(sp_ref, dp_ref, h2_ref, wr3t_ref, b3_ref, eps_ref, z_ref):
    s = sp_ref[0, : _N, :] + sp_ref[1, : _N, :]
    pre = (s / _deg_from(dp_ref) + b3_ref[...]
           + jnp.dot(h2_ref[...], wr3t_ref[...],
                     preferred_element_type=jnp.float32))
    mean = pre[:, : 64]
    log_std = pre[:, 64:]
    z_ref[...] = mean + jnp.exp(log_std) * eps_ref[...]


_tc0 = pl.pallas_call(
    _tc0_body, out_shape=jax.ShapeDtypeStruct((_N, _D), jnp.float32))
_tc1 = pl.pallas_call(
    _tc1_body, out_shape=jax.ShapeDtypeStruct((_N, _D), jnp.float32))
_tc2 = pl.pallas_call(
    _tc2_body, out_shape=[jax.ShapeDtypeStruct((_N, 256), jnp.float32),
                          jax.ShapeDtypeStruct((_N, _D), jnp.float32)])
_tc3 = pl.pallas_call(
    _tc3_body, out_shape=jax.ShapeDtypeStruct((_N, 64), jnp.float32))


def kernel(x, edge_index, Wl1, Wr1, b1, g1, be1, Wl2, Wr2, b2, g2, be2,
           Wl3, Wr3, b3, eps):
    src = edge_index[0].reshape(_NW, _NCHUNK, _CH)
    dst = edge_index[1].reshape(_NW, _NCHUNK, _CH)

    y1 = _tc0(x, Wl1.T)
    (d1,) = _deg(dst)
    (s1,) = _seg(y1, src, dst)
    h1 = _tc1(s1, d1, x, Wr1.T, b1[None, :], g1[None, :], be1[None, :])
    (s2,) = _seg(h1, src, dst)
    h2, y3 = _tc2(s2, d1, h1, Wl2.T, Wr2.T, b2[None, :], g2[None, :],
                  be2[None, :], Wl3.T)
    (s3,) = _seg(y3, src, dst)
    z = _tc3(s3, d1, h2, Wr3.T, b3[None, :], eps)
    return z


# speedup vs baseline: 12.6531x; 1.3285x over previous
import functools

import jax
import jax.numpy as jnp
from jax import lax
from jax.experimental import pallas as pl
from jax.experimental.pallas import tpu as pltpu
from jax.experimental.pallas import tpu_sc as plsc

_N = 10000
_E = 320000
_D = 128
_NC = 2
_NS = 16
_NW = _NC * _NS
_EPT = _E // _NW
_CH = 125
_NCHUNK = _EPT // _CH
_NPAD = 10240
_RPT = _NPAD // _NS


_G = 16
_NGRP = _NCHUNK // _G


def _seg_body(y_hbm, src_hbm, dst_hbm, agg_out, srcA, dstA, srcB, dstB,
              rows0, rows1, acc, sem0, sem1, semi):
    c = lax.axis_index("c")
    s = lax.axis_index("s")
    blk = c * _NS + s

    def _zrow(i, _):
        for k in range(8):
            rows0[i, pl.ds(k * 16, 16)] = jnp.zeros((16,), jnp.float32)
        return 0
    lax.fori_loop(0, _CH, _zrow, 0)
    base = s * _RPT
    for t in range(5):
        pltpu.sync_copy(rows0, acc.at[pl.ds(base + t * _CH, _CH)])
    pltpu.sync_copy(rows0.at[pl.ds(0, _RPT - 5 * _CH)],
                    acc.at[pl.ds(base + 5 * _CH, _RPT - 5 * _CH)])
    plsc.subcore_barrier()

    idx = [(srcA, dstA), (srcB, dstB)]
    pltpu.sync_copy(src_hbm.at[blk, pl.ds(0, _G)], srcA)
    pltpu.sync_copy(dst_hbm.at[blk, pl.ds(0, _G)], dstA)
    cp0 = pltpu.async_copy(y_hbm.at[srcA.at[0]], rows0, sem0)
    cp1 = pltpu.async_copy(y_hbm.at[srcA.at[1]], rows1, sem1)
    for g in range(_NGRP):
        srcg, dstg = idx[g % 2]
        srcn, dstn = idx[(g + 1) % 2]
        if g + 1 < _NGRP:
            cpi0 = pltpu.async_copy(src_hbm.at[blk, pl.ds((g + 1) * _G, _G)],
                                    srcn, semi)
            cpi1 = pltpu.async_copy(dst_hbm.at[blk, pl.ds((g + 1) * _G, _G)],
                                    dstn, semi)

        def _step(i, _):
            j0 = i * 2
            cp0 = pltpu.make_async_copy(y_hbm.at[srcg.at[j0]], rows0, sem0)
            cp1 = pltpu.make_async_copy(y_hbm.at[srcg.at[j0 + 1]], rows1,
                                        sem1)
            cp0.wait()
            pltpu.sync_copy(rows0, acc.at[dstg.at[j0]], add=True)
            pltpu.async_copy(y_hbm.at[srcg.at[j0 + 2]], rows0, sem0)
            cp1.wait()
            pltpu.sync_copy(rows1, acc.at[dstg.at[j0 + 1]], add=True)
            pltpu.async_copy(y_hbm.at[srcg.at[j0 + 3]], rows1, sem1)
            return 0
        lax.fori_loop(0, _G // 2 - 1, _step, 0)

        if g + 1 < _NGRP:
            cpi0.wait()
            cpi1.wait()
        pltpu.make_async_copy(y_hbm.at[srcg.at[_G - 2]], rows0, sem0).wait()
        pltpu.sync_copy(rows0, acc.at[dstg.at[_G - 2]], add=True)
        if g + 1 < _NGRP:
            pltpu.async_copy(y_hbm.at[srcn.at[0]], rows0, sem0)
        pltpu.make_async_copy(y_hbm.at[srcg.at[_G - 1]], rows1, sem1).wait()
        pltpu.sync_copy(rows1, acc.at[dstg.at[_G - 1]], add=True)
        if g + 1 < _NGRP:
            pltpu.async_copy(y_hbm.at[srcn.at[1]], rows1, sem1)

    plsc.subcore_barrier()

    for t in range(5):
        r = base + t * _CH
        pltpu.sync_copy(acc.at[pl.ds(r, _CH)], rows0)
        pltpu.sync_copy(rows0, agg_out.at[c, pl.ds(r, _CH)])
    rem = _RPT - 5 * _CH
    pltpu.sync_copy(acc.at[pl.ds(base + 5 * _CH, rem)],
                    rows0.at[pl.ds(0, rem)])
    pltpu.sync_copy(rows0.at[pl.ds(0, rem)],
                    agg_out.at[c, pl.ds(base + 5 * _CH, rem)])


def _deg_body(dst_hbm, deg_out, dstv, ones, dacc):
    c = lax.axis_index("c")
    s = lax.axis_index("s")
    blk = c * _NS + s

    pltpu.sync_copy(dst_hbm.at[blk], dstv)

    def _fill(val):
        def _row(i, _):
            ones[i, :] = jnp.full((16,), val, jnp.float32)
            return 0
        lax.fori_loop(0, 128, _row, 0)
    _fill(0.0)
    for t in range(_RPT // 128):
        pltpu.sync_copy(ones, dacc.at[pl.ds(s * _RPT + t * 128, 128)])
    _fill(1.0)
    plsc.subcore_barrier()

    o = ones.at[pl.ds(0, _CH)]
    def _step(j, _):
        pltpu.sync_copy(o, dacc.at[dstv.at[j]], add=True)
        return 0
    lax.fori_loop(0, _NCHUNK, _step, 0)

    plsc.subcore_barrier()

    for t in range(_RPT // 128):
        r = s * _RPT + t * 128
        pltpu.sync_copy(dacc.at[pl.ds(r, 128)], ones)
        pltpu.sync_copy(ones, deg_out.at[c, pl.ds(r, 128)])


def _make_seg():
    mesh = plsc.VectorSubcoreMesh(core_axis_name="c", subcore_axis_name="s")
    return pl.kernel(
        _seg_body,
        out_type=[jax.ShapeDtypeStruct((_NC, _NPAD, _D), jnp.float32)],
        mesh=mesh,
        scratch_types=[
            pltpu.VMEM((_G, _CH), jnp.int32),
            pltpu.VMEM((_G, _CH), jnp.int32),
            pltpu.VMEM((_G, _CH), jnp.int32),
            pltpu.VMEM((_G, _CH), jnp.int32),
            pltpu.VMEM((_CH, _D), jnp.float32),
            pltpu.VMEM((_CH, _D), jnp.float32),
            pltpu.VMEM_SHARED((_NPAD, _D), jnp.float32),
            pltpu.SemaphoreType.DMA,
            pltpu.SemaphoreType.DMA,
            pltpu.SemaphoreType.DMA,
        ],
        compiler_params=pltpu.CompilerParams(use_tc_tiling_on_sc=False),
        name="seg_sum",
    )


def _make_deg():
    mesh = plsc.VectorSubcoreMesh(core_axis_name="c", subcore_axis_name="s")
    return pl.kernel(
        _deg_body,
        out_type=[jax.ShapeDtypeStruct((_NC, _NPAD, 16), jnp.float32)],
        mesh=mesh,
        scratch_types=[
            pltpu.VMEM((_NCHUNK, _CH), jnp.int32),
            pltpu.VMEM((128, 16), jnp.float32),
            pltpu.VMEM_SHARED((_NPAD, 16), jnp.float32),
        ],
        compiler_params=pltpu.CompilerParams(use_tc_tiling_on_sc=False),
        name="deg_sum",
    )


_seg = _make_seg()
_deg = _make_deg()


def _tc0_body(x_ref, wl1t_ref, y_ref):
    y_ref[...] = jnp.dot(x_ref[...], wl1t_ref[...],
                         preferred_element_type=jnp.float32)


def _bn(h, g, be):
    m = jnp.mean(h, axis=0, keepdims=True)
    v = jnp.mean((h - m) * (h - m), axis=0, keepdims=True)
    return (h - m) / jnp.sqrt(v + 1e-5) * g + be


def _deg_from(dp_ref):
    deg = dp_ref[0, : _N, 0:1] + dp_ref[1, : _N, 0:1]
    return jnp.maximum(deg, 1.0)


def _tc1_body(sp_ref, dp_ref, x_ref, wr1t_ref, b1_ref, g1_ref, be1_ref,
              h1_ref):
    s = sp_ref[0, : _N, :] + sp_ref[1, : _N, :]
    pre = (s / _deg_from(dp_ref) + b1_ref[...]
           + jnp.dot(x_ref[...], wr1t_ref[...],
                     preferred_element_type=jnp.float32))
    h = jnp.maximum(pre, 0.0)
    h1_ref[...] = _bn(h, g1_ref[...], be1_ref[...])


def _tc2_body(sp_ref, dp_ref, h1_ref, wl2t_ref, wr2t_ref, b2_ref, g2_ref,
              be2_ref, wl3t_ref, h2_ref, y3_ref):
    s = sp_ref[0, : _N, :] + sp_ref[1, : _N, :]
    agg = s / _deg_from(dp_ref)
    pre = (jnp.dot(agg, wl2t_ref[...], preferred_element_type=jnp.float32)
           + b2_ref[...]
           + jnp.dot(h1_ref[...], wr2t_ref[...],
                     preferred_element_type=jnp.float32))
    h = jnp.maximum(pre, 0.0)
    h2 = _bn(h, g2_ref[...], be2_ref[...])
    h2_ref[...] = h2
    y3_ref[...] = jnp.dot(h2, wl3t_ref[...],
                          preferred_element_type=jnp.float32)


def _tc3_body(sp_ref, dp_ref, h2_ref, wr3t_ref, b3_ref, eps_ref, z_ref):
    s = sp_ref[0, : _N, :] + sp_ref[1, : _N, :]
    pre = (s / _deg_from(dp_ref) + b3_ref[...]
           + jnp.dot(h2_ref[...], wr3t_ref[...],
                     preferred_element_type=jnp.float32))
    mean = pre[:, : 64]
    log_std = pre[:, 64:]
    z_ref[...] = mean + jnp.exp(log_std) * eps_ref[...]


_tc0 = pl.pallas_call(
    _tc0_body, out_shape=jax.ShapeDtypeStruct((_N, _D), jnp.float32))
_tc1 = pl.pallas_call(
    _tc1_body, out_shape=jax.ShapeDtypeStruct((_N, _D), jnp.float32))
_tc2 = pl.pallas_call(
    _tc2_body, out_shape=[jax.ShapeDtypeStruct((_N, 256), jnp.float32),
                          jax.ShapeDtypeStruct((_N, _D), jnp.float32)])
_tc3 = pl.pallas_call(
    _tc3_body, out_shape=jax.ShapeDtypeStruct((_N, 64), jnp.float32))


def kernel(x, edge_index, Wl1, Wr1, b1, g1, be1, Wl2, Wr2, b2, g2, be2,
           Wl3, Wr3, b3, eps):
    src = edge_index[0].reshape(_NW, _NCHUNK, _CH)
    dst = edge_index[1].reshape(_NW, _NCHUNK, _CH)

    y1 = _tc0(x, Wl1.T)
    (d1,) = _deg(dst)
    (s1,) = _seg(y1, src, dst)
    h1 = _tc1(s1, d1, x, Wr1.T, b1[None, :], g1[None, :], be1[None, :])
    (s2,) = _seg(h1, src, dst)
    h2, y3 = _tc2(s2, d1, h1, Wl2.T, Wr2.T, b2[None, :], g2[None, :],
                  be2[None, :], Wl3.T)
    (s3,) = _seg(y3, src, dst)
    z = _tc3(s3, d1, h2, Wr3.T, b3[None, :], eps)
    return z
